# baseline (speedup 1.0000x reference)
import math

import jax
import jax.numpy as jnp
from jax.experimental import pallas as pl

N = 20000
C = 81
CM1 = C - 1
K = 500
NPAD = 20480
NROW = 160
NLANE = 128
KPAD = 512
SCORE_THRESH = 0.05
NMS_THRESH = 0.5
DET_PER_IMG = 100
IMG_W = 1333.0
IMG_H = 800.0
WX, WY, WW, WH = 10.0, 10.0, 5.0, 5.0
BBOX_XFORM_CLIP = math.log(1000.0 / 16.0)


def _softmax_mask_kernel(logits_ref, out_ref):
    x = logits_ref[...]
    m = jnp.max(x, axis=1, keepdims=True)
    e = jnp.exp(x - m)
    s = jnp.sum(e, axis=1, keepdims=True)
    p = e / s
    out_ref[...] = jnp.where(p > SCORE_THRESH, p, -jnp.inf)


def _keys_from_scores(s):
    k = jax.lax.bitcast_convert_type(s, jnp.int32)
    return jnp.where(s > 0.0, k, 0)


CGRP = 10


def _select_thresh_kernel(scores_ref, p_ref):
    keys = _keys_from_scores(scores_ref[...])
    kcnt = jnp.float32(K)

    def body(t, p):
        x = p | (jnp.int32(1) << (29 - t))
        cnt = jnp.sum((keys >= x).astype(jnp.float32), axis=(1, 2),
                      keepdims=True)
        return jnp.where(cnt >= kcnt, x, p)

    p = jax.lax.fori_loop(0, 30, body, jnp.zeros((CGRP, 1, 1), jnp.int32))
    p_ref[...] = jnp.broadcast_to(p, (CGRP, 1, NLANE))


def _select_compact_kernel(scores_ref, p_ref, sout_ref, iout_ref):
    keys = _keys_from_scores(scores_ref[0])
    p = p_ref[0]

    sel_gt = keys > p
    sel_eq = keys == p
    n_gt = jnp.sum(sel_gt.astype(jnp.float32))
    m = jnp.float32(K) - n_gt

    lower128 = (jax.lax.broadcasted_iota(jnp.int32, (NLANE, NLANE), 0)
                < jax.lax.broadcasted_iota(jnp.int32, (NLANE, NLANE), 1)
                ).astype(jnp.bfloat16)
    lower160 = (jax.lax.broadcasted_iota(jnp.int32, (NROW, NROW), 1)
                < jax.lax.broadcasted_iota(jnp.int32, (NROW, NROW), 0)
                ).astype(jnp.bfloat16)

    def excl_prefix(mask):
        mb = mask.astype(jnp.bfloat16)
        pref_l = jax.lax.dot_general(
            mb, lower128, (((1,), (0,)), ((), ())),
            preferred_element_type=jnp.float32)
        tot = jnp.sum(mask.astype(jnp.float32), axis=1, keepdims=True)
        row_pref = jax.lax.dot_general(
            lower160, tot.astype(jnp.bfloat16), (((1,), (0,)), ((), ())),
            preferred_element_type=jnp.float32)
        return pref_l, row_pref, tot

    eq_pl, eq_rp, _ = excl_prefix(sel_eq)
    tie_rank = eq_rp + eq_pl
    sel = sel_gt | (sel_eq & (tie_rank < m))

    pref_l, row_pref, tot = excl_prefix(sel)

    r = jax.lax.broadcasted_iota(jnp.int32, (KPAD, 1), 0).astype(
        jnp.float32)
    rvec = jnp.reshape(row_pref, (1, NROW))
    tvec = jnp.reshape(tot, (1, NROW))
    g = ((rvec <= r) & (r < rvec + tvec)).astype(jnp.bfloat16)

    g32 = g.astype(jnp.float32)
    rp_hi = jnp.floor(row_pref * (1.0 / 256.0))
    rp_lo = row_pref - rp_hi * 256.0
    off = r - (jax.lax.dot_general(
        g32, rp_hi, (((1,), (0,)), ((), ())),
        preferred_element_type=jnp.float32) * 256.0
        + jax.lax.dot_general(
            g32, rp_lo, (((1,), (0,)), ((), ())),
            preferred_element_type=jnp.float32))
    row_iota = jax.lax.broadcasted_iota(jnp.int32, (NROW, 1), 0).astype(
        jnp.float32)
    row_id = jax.lax.dot_general(
        g32, row_iota, (((1,), (0,)), ((), ())),
        preferred_element_type=jnp.float32)

    b0 = (keys & 255).astype(jnp.bfloat16)
    b1 = ((keys >> 8) & 255).astype(jnp.bfloat16)
    b2 = ((keys >> 16) & 255).astype(jnp.bfloat16)
    b3 = (keys >> 24).astype(jnp.bfloat16)

    def row_gather(col):
        return jax.lax.dot_general(
            g32, col.astype(jnp.float32), (((1,), (0,)), ((), ())),
            preferred_element_type=jnp.float32)

    gp = row_gather(pref_l)
    gs = row_gather(sel)
    h = ((gp == off) & (gs > 0.5)).astype(jnp.float32)
    lane_iota = jax.lax.broadcasted_iota(jnp.int32, (1, NLANE), 1).astype(
        jnp.float32)
    lane = jnp.sum(h * lane_iota, axis=1, keepdims=True)
    v0 = jnp.sum(h * row_gather(b0), axis=1, keepdims=True)
    v1 = jnp.sum(h * row_gather(b1), axis=1, keepdims=True)
    v2 = jnp.sum(h * row_gather(b2), axis=1, keepdims=True)
    v3 = jnp.sum(h * row_gather(b3), axis=1, keepdims=True)

    key = (v3.astype(jnp.int32) * 16777216 + v2.astype(jnp.int32) * 65536
           + v1.astype(jnp.int32) * 256 + v0.astype(jnp.int32))
    score = jnp.where(key > 0,
                      jax.lax.bitcast_convert_type(key, jnp.float32),
                      -jnp.inf)
    gidx = row_id * jnp.float32(NLANE) + lane

    sout_ref[0] = jnp.reshape(score, (1, KPAD))
    iout_ref[0] = jnp.reshape(gidx, (1, KPAD))


def _nms_kernel(s_ref, i_ref, rel_ref, prop_ref, sout_ref, box_ref):
    s = s_ref[0]
    ix = i_ref[0]
    r = rel_ref[0]
    p = prop_ref[0]

    w = p[2:3] - p[0:1] + 1.0
    h = p[3:4] - p[1:2] + 1.0
    cx = p[0:1] + 0.5 * w
    cy = p[1:2] + 0.5 * h
    dx = r[0:1] / WX
    dy = r[1:2] / WY
    dw = jnp.minimum(r[2:3] / WW, BBOX_XFORM_CLIP)
    dh = jnp.minimum(r[3:4] / WH, BBOX_XFORM_CLIP)
    pcx = dx * w + cx
    pcy = dy * h + cy
    pw = jnp.exp(dw) * w
    ph = jnp.exp(dh) * h
    x1 = jnp.clip(pcx - 0.5 * pw, 0.0, IMG_W - 1.0)
    y1 = jnp.clip(pcy - 0.5 * ph, 0.0, IMG_H - 1.0)
    x2 = jnp.clip(pcx + 0.5 * pw - 1.0, 0.0, IMG_W - 1.0)
    y2 = jnp.clip(pcy + 0.5 * ph - 1.0, 0.0, IMG_H - 1.0)

    area = (x2 - x1) * (y2 - y1)
    x1t = jnp.reshape(x1, (K, 1))
    y1t = jnp.reshape(y1, (K, 1))
    x2t = jnp.reshape(x2, (K, 1))
    y2t = jnp.reshape(y2, (K, 1))
    areat = jnp.reshape(area, (K, 1))
    xx1 = jnp.maximum(x1t, x1)
    yy1 = jnp.maximum(y1t, y1)
    xx2 = jnp.minimum(x2t, x2)
    yy2 = jnp.minimum(y2t, y2)
    inter = jnp.maximum(xx2 - xx1, 0.0) * jnp.maximum(yy2 - yy1, 0.0)
    union = areat + area - inter
    iou = inter / jnp.maximum(union, 1e-9)

    st = jnp.reshape(s, (K, 1))
    ixt = jnp.reshape(ix, (K, 1))
    pre = (st > s) | ((st == s) & (ixt < ix))
    sup = jnp.where((iou > NMS_THRESH) & pre, 1.0, 0.0)

    validf = jnp.where(s > 0.0, 1.0, 0.0)

    def cond(carry):
        return carry[1]

    def body(carry):
        keep, _ = carry
        suppressed = jax.lax.dot_general(
            keep, sup, (((1,), (0,)), ((), ())),
            preferred_element_type=jnp.float32)
        newk = validf * jnp.where(suppressed > 0.0, 0.0, 1.0)
        return (newk, jnp.any(newk != keep))

    keep, _ = jax.lax.while_loop(cond, body, (validf, jnp.bool_(True)))

    sout_ref[0] = jnp.where(keep > 0.0, s, -jnp.inf)
    box_ref[0] = jnp.concatenate([x1, y1, x2, y2], axis=0)


def kernel(class_logits, box_regression, features, proposal_boxes):
    nblk = 2000
    masked = pl.pallas_call(
        _softmax_mask_kernel,
        grid=(N // nblk,),
        in_specs=[pl.BlockSpec((nblk, C), lambda i: (i, 0))],
        out_specs=pl.BlockSpec((nblk, C), lambda i: (i, 0)),
        out_shape=jax.ShapeDtypeStruct((N, C), jnp.float32),
    )(class_logits)

    cls_scores = masked.T[1:]
    scores3 = jnp.pad(cls_scores, ((0, 0), (0, NPAD - N)),
                      constant_values=-jnp.inf).reshape(CM1, NROW, NLANE)

    pthr = pl.pallas_call(
        _select_thresh_kernel,
        grid=(CM1 // CGRP,),
        in_specs=[pl.BlockSpec((CGRP, NROW, NLANE), lambda i: (i, 0, 0))],
        out_specs=pl.BlockSpec((CGRP, 1, NLANE), lambda i: (i, 0, 0)),
        out_shape=jax.ShapeDtypeStruct((CM1, 1, NLANE), jnp.int32),
    )(scores3)

    top_sp, idxf = pl.pallas_call(
        _select_compact_kernel,
        grid=(CM1,),
        in_specs=[
            pl.BlockSpec((1, NROW, NLANE), lambda c: (c, 0, 0)),
            pl.BlockSpec((1, 1, NLANE), lambda c: (c, 0, 0)),
        ],
        out_specs=[
            pl.BlockSpec((1, 1, KPAD), lambda c: (c, 0, 0)),
            pl.BlockSpec((1, 1, KPAD), lambda c: (c, 0, 0)),
        ],
        out_shape=[
            jax.ShapeDtypeStruct((CM1, 1, KPAD), jnp.float32),
            jax.ShapeDtypeStruct((CM1, 1, KPAD), jnp.float32),
        ],
    )(scores3, pthr)

    top_s = top_sp[:, 0, :K]
    idxf3 = idxf[:, :, :K]
    idx = idxf[:, 0, :K].astype(jnp.int32)

    rel = box_regression.reshape(N, C, 4)
    cls_ids = jnp.arange(1, C)[:, None]
    rel_t = rel[idx, cls_ids].transpose(0, 2, 1)
    prop_t = proposal_boxes[idx].transpose(0, 2, 1)

    s_out, box_t = pl.pallas_call(
        _nms_kernel,
        grid=(CM1,),
        in_specs=[
            pl.BlockSpec((1, 1, K), lambda c: (c, 0, 0)),
            pl.BlockSpec((1, 1, K), lambda c: (c, 0, 0)),
            pl.BlockSpec((1, 4, K), lambda c: (c, 0, 0)),
            pl.BlockSpec((1, 4, K), lambda c: (c, 0, 0)),
        ],
        out_specs=[
            pl.BlockSpec((1, 1, K), lambda c: (c, 0, 0)),
            pl.BlockSpec((1, 4, K), lambda c: (c, 0, 0)),
        ],
        out_shape=[
            jax.ShapeDtypeStruct((CM1, 1, K), jnp.float32),
            jax.ShapeDtypeStruct((CM1, 4, K), jnp.float32),
        ],
    )(top_s[:, None, :], idxf3, rel_t, prop_t)

    flat_s = s_out.reshape(-1)
    flat_b = box_t.transpose(0, 2, 1).reshape(-1, 4)
    flat_idx = idx.reshape(-1)
    labels = jnp.broadcast_to(jnp.arange(1, C)[:, None], (CM1, K)).reshape(-1)

    top_s2, top_i = jax.lax.top_k(flat_s, DET_PER_IMG)
    final_b = flat_b[top_i]
    final_l = labels[top_i].astype(jnp.float32)
    final_feat = features[flat_idx[top_i]]
    final_s = jnp.where(jnp.isfinite(top_s2), top_s2, 0.0)
    return jnp.concatenate(
        [final_b, final_s[:, None], final_l[:, None], final_feat], axis=1)


# speedup vs baseline: 2.8479x; 2.8479x over previous
import math

import jax
import jax.numpy as jnp
from jax.experimental import pallas as pl

N = 20000
C = 81
CM1 = C - 1
K = 500
NPAD = 20480
NROW = 160
NLANE = 128
KPAD = 512
SCORE_THRESH = 0.05
NMS_THRESH = 0.5
DET_PER_IMG = 100
IMG_W = 1333.0
IMG_H = 800.0
WX, WY, WW, WH = 10.0, 10.0, 5.0, 5.0
BBOX_XFORM_CLIP = math.log(1000.0 / 16.0)


def _softmax_mask_kernel(logits_ref, out_ref):
    x = logits_ref[...]
    m = jnp.max(x, axis=1, keepdims=True)
    e = jnp.exp(x - m)
    s = jnp.sum(e, axis=1, keepdims=True)
    p = e / s
    out_ref[...] = jnp.where(p > SCORE_THRESH, p, -jnp.inf)


def _keys_from_scores(s):
    k = jax.lax.bitcast_convert_type(s, jnp.int32)
    return jnp.where(s > 0.0, k, 0)


CGRP = 10


def _select_thresh_kernel(scores_ref, p_ref):
    keys = _keys_from_scores(scores_ref[...])
    kcnt = jnp.float32(K)

    def body(t, p):
        x = p | (jnp.int32(1) << (29 - t))
        cnt = jnp.sum((keys >= x).astype(jnp.float32), axis=(1, 2),
                      keepdims=True)
        return jnp.where(cnt >= kcnt, x, p)

    p = jax.lax.fori_loop(0, 30, body, jnp.zeros((CGRP, 1, 1), jnp.int32))
    p_ref[...] = jnp.broadcast_to(p, (CGRP, 1, NLANE))


def _select_compact_kernel(scores_ref, p_ref, sout_ref, iout_ref):
    keys = _keys_from_scores(scores_ref[0])
    p = p_ref[0]

    sel_gt = keys > p
    sel_eq = keys == p
    n_gt = jnp.sum(sel_gt.astype(jnp.float32))
    m = jnp.float32(K) - n_gt

    lower128 = (jax.lax.broadcasted_iota(jnp.int32, (NLANE, NLANE), 0)
                < jax.lax.broadcasted_iota(jnp.int32, (NLANE, NLANE), 1)
                ).astype(jnp.bfloat16)
    lower160 = (jax.lax.broadcasted_iota(jnp.int32, (NROW, NROW), 1)
                < jax.lax.broadcasted_iota(jnp.int32, (NROW, NROW), 0)
                ).astype(jnp.bfloat16)

    def excl_prefix(mask):
        mb = mask.astype(jnp.bfloat16)
        pref_l = jax.lax.dot_general(
            mb, lower128, (((1,), (0,)), ((), ())),
            preferred_element_type=jnp.float32)
        tot = jnp.sum(mask.astype(jnp.float32), axis=1, keepdims=True)
        row_pref = jax.lax.dot_general(
            lower160, tot.astype(jnp.bfloat16), (((1,), (0,)), ((), ())),
            preferred_element_type=jnp.float32)
        return pref_l, row_pref, tot

    eq_pl, eq_rp, _ = excl_prefix(sel_eq)
    tie_rank = eq_rp + eq_pl
    sel = sel_gt | (sel_eq & (tie_rank < m))

    pref_l, row_pref, tot = excl_prefix(sel)

    r = jax.lax.broadcasted_iota(jnp.int32, (KPAD, 1), 0).astype(
        jnp.float32)
    r_row = jax.lax.broadcasted_iota(jnp.int32, (1, KPAD), 1).astype(
        jnp.float32)
    gt = ((row_pref <= r_row) &
          (r_row < row_pref + tot)).astype(jnp.bfloat16)

    rp_hi = jnp.floor(row_pref * (1.0 / 256.0))
    rp_lo = row_pref - rp_hi * 256.0
    row_iota = jax.lax.broadcasted_iota(jnp.int32, (NROW, 1), 0).astype(
        jnp.float32)

    b0 = (keys & 255).astype(jnp.bfloat16)
    b1 = ((keys >> 8) & 255).astype(jnp.bfloat16)
    b2 = ((keys >> 16) & 255).astype(jnp.bfloat16)
    b3 = (keys >> 24).astype(jnp.bfloat16)
    d = jnp.concatenate(
        [pref_l.astype(jnp.bfloat16), sel.astype(jnp.bfloat16),
         b0, b1, b2, b3,
         jnp.broadcast_to(rp_hi.astype(jnp.bfloat16), (NROW, NLANE)),
         jnp.broadcast_to(rp_lo.astype(jnp.bfloat16), (NROW, NLANE)),
         jnp.broadcast_to(row_iota.astype(jnp.bfloat16), (NROW, NLANE)),
         ], axis=1)
    gath = jax.lax.dot_general(
        gt, d, (((0,), (0,)), ((), ())),
        preferred_element_type=jnp.float32)

    gp = gath[:, 0:NLANE]
    gs = gath[:, NLANE:2 * NLANE]
    off = r - (gath[:, 6 * NLANE:6 * NLANE + 1] * 256.0
               + gath[:, 7 * NLANE:7 * NLANE + 1])
    row_id = gath[:, 8 * NLANE:8 * NLANE + 1]
    h = ((gp == off) & (gs > 0.5)).astype(jnp.float32)
    lane_iota = jax.lax.broadcasted_iota(jnp.int32, (1, NLANE), 1).astype(
        jnp.float32)
    lane = jnp.sum(h * lane_iota, axis=1, keepdims=True)
    v0 = jnp.sum(h * gath[:, 2 * NLANE:3 * NLANE], axis=1, keepdims=True)
    v1 = jnp.sum(h * gath[:, 3 * NLANE:4 * NLANE], axis=1, keepdims=True)
    v2 = jnp.sum(h * gath[:, 4 * NLANE:5 * NLANE], axis=1, keepdims=True)
    v3 = jnp.sum(h * gath[:, 5 * NLANE:6 * NLANE], axis=1, keepdims=True)

    key = (v3.astype(jnp.int32) * 16777216 + v2.astype(jnp.int32) * 65536
           + v1.astype(jnp.int32) * 256 + v0.astype(jnp.int32))
    score = jnp.where(key > 0,
                      jax.lax.bitcast_convert_type(key, jnp.float32),
                      -jnp.inf)
    gidx = row_id * jnp.float32(NLANE) + lane

    sout_ref[0] = score
    iout_ref[0] = gidx


def _nms_kernel(s_ref, i_ref, rel_ref, prop_ref, sout_ref, box_ref):
    s = s_ref[0]
    ix = i_ref[0]
    r = rel_ref[0]
    p = prop_ref[0]

    w = p[2:3] - p[0:1] + 1.0
    h = p[3:4] - p[1:2] + 1.0
    cx = p[0:1] + 0.5 * w
    cy = p[1:2] + 0.5 * h
    dx = r[0:1] / WX
    dy = r[1:2] / WY
    dw = jnp.minimum(r[2:3] / WW, BBOX_XFORM_CLIP)
    dh = jnp.minimum(r[3:4] / WH, BBOX_XFORM_CLIP)
    pcx = dx * w + cx
    pcy = dy * h + cy
    pw = jnp.exp(dw) * w
    ph = jnp.exp(dh) * h
    x1 = jnp.clip(pcx - 0.5 * pw, 0.0, IMG_W - 1.0)
    y1 = jnp.clip(pcy - 0.5 * ph, 0.0, IMG_H - 1.0)
    x2 = jnp.clip(pcx + 0.5 * pw - 1.0, 0.0, IMG_W - 1.0)
    y2 = jnp.clip(pcy + 0.5 * ph - 1.0, 0.0, IMG_H - 1.0)

    area = (x2 - x1) * (y2 - y1)
    x1t = jnp.reshape(x1, (K, 1))
    y1t = jnp.reshape(y1, (K, 1))
    x2t = jnp.reshape(x2, (K, 1))
    y2t = jnp.reshape(y2, (K, 1))
    areat = jnp.reshape(area, (K, 1))
    xx1 = jnp.maximum(x1t, x1)
    yy1 = jnp.maximum(y1t, y1)
    xx2 = jnp.minimum(x2t, x2)
    yy2 = jnp.minimum(y2t, y2)
    inter = jnp.maximum(xx2 - xx1, 0.0) * jnp.maximum(yy2 - yy1, 0.0)
    union = areat + area - inter
    iou = inter / jnp.maximum(union, 1e-9)

    st = jnp.reshape(s, (K, 1))
    ixt = jnp.reshape(ix, (K, 1))
    pre = (st > s) | ((st == s) & (ixt < ix))
    sup = jnp.where((iou > NMS_THRESH) & pre, 1.0, 0.0)

    validf = jnp.where(s > 0.0, 1.0, 0.0)

    def cond(carry):
        return carry[1]

    def body(carry):
        keep, _ = carry
        suppressed = jax.lax.dot_general(
            keep, sup, (((1,), (0,)), ((), ())),
            preferred_element_type=jnp.float32)
        newk = validf * jnp.where(suppressed > 0.0, 0.0, 1.0)
        return (newk, jnp.any(newk != keep))

    keep, _ = jax.lax.while_loop(cond, body, (validf, jnp.bool_(True)))

    sout_ref[0] = jnp.where(keep > 0.0, s, -jnp.inf)
    box_ref[0] = jnp.concatenate([x1, y1, x2, y2], axis=0)


def kernel(class_logits, box_regression, features, proposal_boxes):
    nblk = 2000
    masked = pl.pallas_call(
        _softmax_mask_kernel,
        grid=(N // nblk,),
        in_specs=[pl.BlockSpec((nblk, C), lambda i: (i, 0))],
        out_specs=pl.BlockSpec((nblk, C), lambda i: (i, 0)),
        out_shape=jax.ShapeDtypeStruct((N, C), jnp.float32),
    )(class_logits)

    cls_scores = masked.T[1:]
    scores3 = jnp.pad(cls_scores, ((0, 0), (0, NPAD - N)),
                      constant_values=-jnp.inf).reshape(CM1, NROW, NLANE)

    pthr = pl.pallas_call(
        _select_thresh_kernel,
        grid=(CM1 // CGRP,),
        in_specs=[pl.BlockSpec((CGRP, NROW, NLANE), lambda i: (i, 0, 0))],
        out_specs=pl.BlockSpec((CGRP, 1, NLANE), lambda i: (i, 0, 0)),
        out_shape=jax.ShapeDtypeStruct((CM1, 1, NLANE), jnp.int32),
    )(scores3)

    top_sp, idxf = pl.pallas_call(
        _select_compact_kernel,
        grid=(CM1,),
        in_specs=[
            pl.BlockSpec((1, NROW, NLANE), lambda c: (c, 0, 0)),
            pl.BlockSpec((1, 1, NLANE), lambda c: (c, 0, 0)),
        ],
        out_specs=[
            pl.BlockSpec((1, KPAD, 1), lambda c: (c, 0, 0)),
            pl.BlockSpec((1, KPAD, 1), lambda c: (c, 0, 0)),
        ],
        out_shape=[
            jax.ShapeDtypeStruct((CM1, KPAD, 1), jnp.float32),
            jax.ShapeDtypeStruct((CM1, KPAD, 1), jnp.float32),
        ],
    )(scores3, pthr)

    top_s = top_sp[:, :K, 0]
    idxf3 = idxf[:, :K, 0][:, None, :]
    idx = idxf[:, :K, 0].astype(jnp.int32)

    rel = box_regression.reshape(N, C, 4)
    cls_ids = jnp.arange(1, C)[:, None]
    rel_t = rel[idx, cls_ids].transpose(0, 2, 1)
    prop_t = proposal_boxes[idx].transpose(0, 2, 1)

    s_out, box_t = pl.pallas_call(
        _nms_kernel,
        grid=(CM1,),
        in_specs=[
            pl.BlockSpec((1, 1, K), lambda c: (c, 0, 0)),
            pl.BlockSpec((1, 1, K), lambda c: (c, 0, 0)),
            pl.BlockSpec((1, 4, K), lambda c: (c, 0, 0)),
            pl.BlockSpec((1, 4, K), lambda c: (c, 0, 0)),
        ],
        out_specs=[
            pl.BlockSpec((1, 1, K), lambda c: (c, 0, 0)),
            pl.BlockSpec((1, 4, K), lambda c: (c, 0, 0)),
        ],
        out_shape=[
            jax.ShapeDtypeStruct((CM1, 1, K), jnp.float32),
            jax.ShapeDtypeStruct((CM1, 4, K), jnp.float32),
        ],
    )(top_s[:, None, :], idxf3, rel_t, prop_t)

    flat_s = s_out.reshape(-1)
    flat_b = box_t.transpose(0, 2, 1).reshape(-1, 4)
    flat_idx = idx.reshape(-1)
    labels = jnp.broadcast_to(jnp.arange(1, C)[:, None], (CM1, K)).reshape(-1)

    top_s2, top_i = jax.lax.top_k(flat_s, DET_PER_IMG)
    final_b = flat_b[top_i]
    final_l = labels[top_i].astype(jnp.float32)
    final_feat = features[flat_idx[top_i]]
    final_s = jnp.where(jnp.isfinite(top_s2), top_s2, 0.0)
    return jnp.concatenate(
        [final_b, final_s[:, None], final_l[:, None], final_feat], axis=1)
